# Initial kernel scaffold; baseline (speedup 1.0000x reference)
import jax
import jax.numpy as jnp
from jax.experimental import pallas as pl


def kernel(x, edge_index, batch, Wl0, bl0, Wr0, g0, be0, Wl1, bl1, Wr1, g1, be1, W1, b1, W2, b2):
    raise NotImplementedError("write your pallas kernel here")


# speedup vs baseline: 4.5818x; 4.5818x over previous
import functools

import jax
import jax.numpy as jnp
from jax import lax
from jax.experimental import pallas as pl
from jax.experimental.pallas import tpu as pltpu
from jax.experimental.pallas import tpu_sc as plsc

N = 10000
E = 320000
D = 128
H = 128
B = 64
OUT = 10

NW = 32
EW = E // NW
C = 80
ROWS_S = N // 16
ROWS_D = N // 10

_mesh = plsc.VectorSubcoreMesh(core_axis_name="c", subcore_axis_name="s")


@functools.partial(
    pl.kernel,
    mesh=_mesh,
    out_type=[
        jax.ShapeDtypeStruct((2 * N, H), jnp.float32),
        jax.ShapeDtypeStruct((2 * N,), jnp.float32),
    ],
    scratch_types=[
        pltpu.VMEM((C,), jnp.int32),
        pltpu.VMEM((C,), jnp.int32),
        pltpu.VMEM((C, H), jnp.float32),
        pltpu.VMEM((C,), jnp.float32),
        pltpu.VMEM((ROWS_D,), jnp.float32),
        pltpu.VMEM_SHARED((N, H), jnp.float32),
        pltpu.VMEM_SHARED((N,), jnp.float32),
        pltpu.SemaphoreType.DMA,
    ],
)
def _sc_agg(x_hbm, src_hbm, dst_hbm, zeros2_hbm, zeros1_hbm, ones_hbm,
            agg_out, deg_out, src_v, dst_v, rows_v, ones_v, deg_tmp, agg_sh,
            deg_sh, sem):
    c = lax.axis_index("c")
    s = lax.axis_index("s")
    wid = s * 2 + c

    pltpu.sync_copy(ones_hbm, ones_v)

    if True:
        @pl.when(s < 10)
        def _():
            pltpu.sync_copy(zeros2_hbm.at[pl.ds(s * ROWS_D, ROWS_D)],
                            agg_sh.at[pl.ds(s * ROWS_D, ROWS_D)])
            pltpu.sync_copy(zeros1_hbm.at[pl.ds(s * ROWS_D, ROWS_D)], deg_tmp)
            pltpu.sync_copy(deg_tmp, deg_sh.at[pl.ds(s * ROWS_D, ROWS_D)])

        plsc.subcore_barrier()

        base = wid * EW

        def step(i, carry):
            off = base + i * C
            pltpu.sync_copy(src_hbm.at[pl.ds(off, C)], src_v)
            pltpu.sync_copy(dst_hbm.at[pl.ds(off, C)], dst_v)
            pltpu.async_copy(x_hbm.at[src_v], rows_v, sem).wait()
            pltpu.sync_copy(rows_v, agg_sh.at[dst_v], add=True)
            pltpu.sync_copy(ones_v, deg_sh.at[dst_v], add=True)
            return carry

        lax.fori_loop(0, EW // C, step, 0)
        plsc.subcore_barrier()

        @pl.when(s < 10)
        def _():
            pltpu.sync_copy(agg_sh.at[pl.ds(s * ROWS_D, ROWS_D)],
                            agg_out.at[pl.ds(c * N + s * ROWS_D, ROWS_D)])
            pltpu.sync_copy(deg_sh.at[pl.ds(s * ROWS_D, ROWS_D)], deg_tmp)
            pltpu.sync_copy(deg_tmp,
                            deg_out.at[pl.ds(c * N + s * ROWS_D, ROWS_D)])


def _layer_math(x, agg_ref, deg_ref, wl_ref, bl_ref, wr_ref, g_ref, be_ref):
    a = agg_ref[0:N, :] + agg_ref[N:2 * N, :]
    d = jnp.maximum(deg_ref[...], 1.0)
    t = (jnp.dot(a / d, wl_ref[...], preferred_element_type=jnp.float32)
         + bl_ref[...]
         + jnp.dot(x, wr_ref[...], preferred_element_type=jnp.float32))
    t = jnp.maximum(t, 0.0)
    mu = jnp.mean(t, axis=0, keepdims=True)
    v = jnp.mean((t - mu) ** 2, axis=0, keepdims=True)
    return (t - mu) * lax.rsqrt(v + 1e-5) * g_ref[...] + be_ref[...]


def _tc_layer1(x_ref, agg_ref, deg_ref, wl_ref, bl_ref, wr_ref, g_ref,
               be_ref, out_ref):
    out_ref[...] = _layer_math(x_ref[...], agg_ref, deg_ref, wl_ref, bl_ref,
                               wr_ref, g_ref, be_ref)


def _tc_layer2_head(h_ref, agg_ref, deg_ref, wl_ref, bl_ref, wr_ref, g_ref,
                    be_ref, bcol_ref, brow_ref, w1_ref, b1_ref, w2_ref,
                    b2_ref, out_ref):
    h2 = _layer_math(h_ref[...], agg_ref, deg_ref, wl_ref, bl_ref, wr_ref,
                     g_ref, be_ref)
    brow = brow_ref[...]
    iota_b = lax.broadcasted_iota(jnp.int32, (B, 1), 0)
    iota_row = lax.broadcasted_iota(jnp.int32, (1, N), 1)

    onehot_t = jnp.where(iota_b == brow, 1.0, 0.0)
    sums = jnp.dot(onehot_t, h2, preferred_element_type=jnp.float32)
    cnt = jnp.dot(onehot_t, jnp.full((N, 1), 1.0, jnp.float32),
                  preferred_element_type=jnp.float32)
    x2 = sums / jnp.maximum(cnt, 1.0)

    firsts = jnp.min(jnp.where(brow >= iota_b, iota_row, N), axis=1,
                     keepdims=True)
    firstsel = jnp.where(jnp.minimum(firsts, N - 1) == iota_row, 1.0, 0.0)
    x3 = jnp.dot(firstsel, h2, preferred_element_type=jnp.float32)

    bcol = bcol_ref[...]

    def seg(b, x1acc):
        row = jnp.max(jnp.where(bcol == b, h2, -jnp.inf), axis=0,
                      keepdims=True)
        return jnp.where(iota_b == b, row, x1acc)

    x1 = lax.fori_loop(0, B, seg, jnp.full((B, H), -jnp.inf, jnp.float32))

    z = jnp.concatenate([x1, x2, x3], axis=1)
    r = jnp.dot(z, w1_ref[...], preferred_element_type=jnp.float32) + b1_ref[...]
    r = jnp.dot(r, w2_ref[...], preferred_element_type=jnp.float32) + b2_ref[...]
    m = jnp.max(r, axis=1, keepdims=True)
    e = r - m
    out_ref[...] = e - jnp.log(jnp.sum(jnp.exp(e), axis=1, keepdims=True))


def kernel(x, edge_index, batch, Wl0, bl0, Wr0, g0, be0, Wl1, bl1, Wr1, g1,
           be1, W1, b1, W2, b2):
    src = edge_index[0]
    dst = edge_index[1]
    zeros2 = jnp.zeros((N, H), jnp.float32)
    zeros1 = jnp.zeros((N,), jnp.float32)
    onesc = jnp.ones((C,), jnp.float32)
    bcol = batch.reshape(N, 1).astype(jnp.int32)
    brow = batch.reshape(1, N).astype(jnp.int32)

    agg1, deg1 = _sc_agg(x, src, dst, zeros2, zeros1, onesc)
    degp = (deg1[0:N] + deg1[N:2 * N]).reshape(N, 1)

    h1 = pl.pallas_call(
        _tc_layer1,
        out_shape=jax.ShapeDtypeStruct((N, H), jnp.float32),
    )(x, agg1, degp, Wl0, bl0.reshape(1, H), Wr0, g0.reshape(1, H),
      be0.reshape(1, H))

    agg2, _ = _sc_agg(h1, src, dst, zeros2, zeros1, onesc)

    out = pl.pallas_call(
        _tc_layer2_head,
        out_shape=jax.ShapeDtypeStruct((B, OUT), jnp.float32),
    )(h1, agg2, degp, Wl1, bl1.reshape(1, H), Wr1, g1.reshape(1, H),
      be1.reshape(1, H), bcol, brow, W1, b1.reshape(1, 3 * H), W2,
      b2.reshape(1, OUT))
    return out


# speedup vs baseline: 7.7753x; 1.6970x over previous
import functools

import jax
import jax.numpy as jnp
from jax import lax
from jax.experimental import pallas as pl
from jax.experimental.pallas import tpu as pltpu
from jax.experimental.pallas import tpu_sc as plsc

N = 10000
E = 320000
D = 128
H = 128
B = 64
OUT = 10

NW = 32
EW = E // NW
C = 80
JW = EW // C
KB = 5
ROWS_D = N // 10

_mesh = plsc.VectorSubcoreMesh(core_axis_name="c", subcore_axis_name="s")


C2 = 40
KB2 = 5
CB = KB2 * C2
STEPS = EW // CB


def _make_sc_agg(compute_deg):
    outs = [jax.ShapeDtypeStruct((2 * N, H), jnp.float32)]
    scratch = [
        pltpu.VMEM((CB,), jnp.int32),
        pltpu.VMEM((CB,), jnp.int32),
        pltpu.VMEM((2 * KB2, C2), jnp.int32),
        pltpu.VMEM((KB2, C2, H), jnp.float32),
        pltpu.VMEM_SHARED((N, H), jnp.float32),
        pltpu.SemaphoreType.DMA,
        pltpu.SemaphoreType.DMA,
        pltpu.SemaphoreType.DMA,
        pltpu.SemaphoreType.DMA,
    ]
    if compute_deg:
        outs.append(jax.ShapeDtypeStruct((2 * N,), jnp.float32))
        scratch += [
            pltpu.VMEM((C2,), jnp.float32),
            pltpu.VMEM((ROWS_D,), jnp.float32),
            pltpu.VMEM_SHARED((N,), jnp.float32),
        ]

    @functools.partial(pl.kernel, mesh=_mesh, out_type=outs,
                       scratch_types=scratch)
    def _sc(x_hbm, src_hbm, dst_hbm, zeros2_hbm, *rest):
        if compute_deg:
            (zeros1_hbm, ones_hbm, agg_out, deg_out, srcb0, srcb1, dstb,
             rows_v, agg_sh, semi0, semi1, sem_g, sem_s, ones_v, deg_tmp,
             deg_sh) = rest
        else:
            (agg_out, srcb0, srcb1, dstb, rows_v, agg_sh, semi0, semi1,
             sem_g, sem_s) = rest
        c = lax.axis_index("c")
        s = lax.axis_index("s")
        wid = s * 2 + c
        base = wid * EW
        semi = (semi0, semi1)
        srcbufs = (srcb0, srcb1)

        def load_idx(step, pp):
            off = base + step * CB
            pltpu.async_copy(src_hbm.at[pl.ds(off, CB)], srcbufs[pp],
                             semi[pp])
            for b in range(KB2):
                pltpu.async_copy(dst_hbm.at[pl.ds(off + b * C2, C2)],
                                 dstb.at[pp * KB2 + b], semi[pp])

        def drain_idx(pp):
            pltpu.make_async_copy(src_hbm.at[pl.ds(0, CB)], srcbufs[pp],
                                  semi[pp]).wait()
            for b in range(KB2):
                pltpu.make_async_copy(dst_hbm.at[pl.ds(0, C2)],
                                      dstb.at[pp * KB2 + b], semi[pp]).wait()

        load_idx(0, 0)
        if compute_deg:
            pltpu.sync_copy(ones_hbm, ones_v)

        @pl.when(s < 10)
        def _():
            pltpu.sync_copy(zeros2_hbm.at[pl.ds(s * ROWS_D, ROWS_D)],
                            agg_sh.at[pl.ds(s * ROWS_D, ROWS_D)])
            if compute_deg:
                pltpu.sync_copy(zeros1_hbm.at[pl.ds(s * ROWS_D, ROWS_D)],
                                deg_tmp)
                pltpu.sync_copy(deg_tmp, deg_sh.at[pl.ds(s * ROWS_D, ROWS_D)])

        plsc.subcore_barrier()

        def outer(i, carry):
            for pp in range(2):
                step = i * 2 + pp

                @pl.when(step + 1 < STEPS)
                def _():
                    load_idx(step + 1, 1 - pp)

                drain_idx(pp)
                hs = []
                for b in range(KB2):
                    hs.append(pltpu.async_copy(
                        x_hbm.at[srcbufs[pp].at[pl.ds(b * C2, C2)]],
                        rows_v.at[b], sem_g))
                for h in hs:
                    h.wait()
                ss = []
                for b in range(KB2):
                    ss.append(pltpu.async_copy(
                        rows_v.at[b], agg_sh.at[dstb.at[pp * KB2 + b]],
                        sem_s, add=True))
                    if compute_deg:
                        ss.append(pltpu.async_copy(
                            ones_v, deg_sh.at[dstb.at[pp * KB2 + b]],
                            sem_s, add=True))
                for h in ss:
                    h.wait()
            return carry

        lax.fori_loop(0, STEPS // 2, outer, 0)
        plsc.subcore_barrier()

        @pl.when(s < 10)
        def _():
            pltpu.sync_copy(agg_sh.at[pl.ds(s * ROWS_D, ROWS_D)],
                            agg_out.at[pl.ds(c * N + s * ROWS_D, ROWS_D)])
            if compute_deg:
                pltpu.sync_copy(deg_sh.at[pl.ds(s * ROWS_D, ROWS_D)],
                                deg_tmp)
                pltpu.sync_copy(
                    deg_tmp, deg_out.at[pl.ds(c * N + s * ROWS_D, ROWS_D)])

    return _sc


_sc_agg_deg = _make_sc_agg(True)
_sc_agg_nodeg = _make_sc_agg(False)


def _layer_math(x, agg_ref, deg_ref, wl_ref, bl_ref, wr_ref, g_ref, be_ref):
    a = agg_ref[0:N, :] + agg_ref[N:2 * N, :]
    d = jnp.maximum(deg_ref[...], 1.0)
    t = (jnp.dot(a / d, wl_ref[...], preferred_element_type=jnp.float32)
         + bl_ref[...]
         + jnp.dot(x, wr_ref[...], preferred_element_type=jnp.float32))
    t = jnp.maximum(t, 0.0)
    mu = jnp.mean(t, axis=0, keepdims=True)
    v = jnp.mean((t - mu) ** 2, axis=0, keepdims=True)
    return (t - mu) * lax.rsqrt(v + 1e-5) * g_ref[...] + be_ref[...]


def _tc_layer1(x_ref, agg_ref, deg_ref, wl_ref, bl_ref, wr_ref, g_ref,
               be_ref, out_ref):
    out_ref[...] = _layer_math(x_ref[...], agg_ref, deg_ref, wl_ref, bl_ref,
                               wr_ref, g_ref, be_ref)


def _tc_layer2_head(h_ref, agg_ref, deg_ref, wl_ref, bl_ref, wr_ref, g_ref,
                    be_ref, bcol_ref, brow_ref, w1_ref, b1_ref, w2_ref,
                    b2_ref, out_ref):
    h2 = _layer_math(h_ref[...], agg_ref, deg_ref, wl_ref, bl_ref, wr_ref,
                     g_ref, be_ref)
    brow = brow_ref[...]
    iota_b = lax.broadcasted_iota(jnp.int32, (B, 1), 0)
    iota_row = lax.broadcasted_iota(jnp.int32, (1, N), 1)

    onehot_t = jnp.where(iota_b == brow, 1.0, 0.0)
    sums = jnp.dot(onehot_t, h2, preferred_element_type=jnp.float32)
    cnt = jnp.dot(onehot_t, jnp.full((N, 1), 1.0, jnp.float32),
                  preferred_element_type=jnp.float32)
    x2 = sums / jnp.maximum(cnt, 1.0)

    firsts = jnp.min(jnp.where(brow >= iota_b, iota_row, N), axis=1,
                     keepdims=True)
    firstsel = jnp.where(jnp.minimum(firsts, N - 1) == iota_row, 1.0, 0.0)
    x3 = jnp.dot(firstsel, h2, preferred_element_type=jnp.float32)

    bcol = bcol_ref[...]

    def seg(b, x1acc):
        row = jnp.max(jnp.where(bcol == b, h2, -jnp.inf), axis=0,
                      keepdims=True)
        return jnp.where(iota_b == b, row, x1acc)

    x1 = lax.fori_loop(0, B, seg, jnp.full((B, H), -jnp.inf, jnp.float32))

    z = jnp.concatenate([x1, x2, x3], axis=1)
    r = jnp.dot(z, w1_ref[...], preferred_element_type=jnp.float32) + b1_ref[...]
    r = jnp.dot(r, w2_ref[...], preferred_element_type=jnp.float32) + b2_ref[...]
    m = jnp.max(r, axis=1, keepdims=True)
    e = r - m
    out_ref[...] = e - jnp.log(jnp.sum(jnp.exp(e), axis=1, keepdims=True))


def kernel(x, edge_index, batch, Wl0, bl0, Wr0, g0, be0, Wl1, bl1, Wr1, g1,
           be1, W1, b1, W2, b2):
    src = edge_index[0]
    dst = edge_index[1]
    zeros2 = jnp.zeros((N, H), jnp.float32)
    zeros1 = jnp.zeros((N,), jnp.float32)
    onesc = jnp.ones((C2,), jnp.float32)
    bcol = batch.reshape(N, 1).astype(jnp.int32)
    brow = batch.reshape(1, N).astype(jnp.int32)

    agg1, deg1 = _sc_agg_deg(x, src, dst, zeros2, zeros1, onesc)
    degp = (deg1[0:N] + deg1[N:2 * N]).reshape(N, 1)

    h1 = pl.pallas_call(
        _tc_layer1,
        out_shape=jax.ShapeDtypeStruct((N, H), jnp.float32),
    )(x, agg1, degp, Wl0, bl0.reshape(1, H), Wr0, g0.reshape(1, H),
      be0.reshape(1, H))

    agg2 = _sc_agg_nodeg(h1, src, dst, zeros2)
    if isinstance(agg2, (list, tuple)):
        agg2 = agg2[0]

    out = pl.pallas_call(
        _tc_layer2_head,
        out_shape=jax.ShapeDtypeStruct((B, OUT), jnp.float32),
    )(h1, agg2, degp, Wl1, bl1.reshape(1, H), Wr1, g1.reshape(1, H),
      be1.reshape(1, H), bcol, brow, W1, b1.reshape(1, 3 * H), W2,
      b2.reshape(1, OUT))
    return out
